# baseline (speedup 1.0000x reference)
import functools

import jax
import jax.numpy as jnp
from jax import lax
from jax.experimental import pallas as pl
from jax.experimental.pallas import tpu as pltpu
from jax.experimental.pallas import tpu_sc as plsc

E = 320000
D = 128
NC = 2
NS = 16
NW = NC * NS
C = 128
NCH = E // C
TMAX = -(-NCH // NW)
NBUF = 2

_mesh = plsc.VectorSubcoreMesh(core_axis_name="c", subcore_axis_name="s")


@functools.partial(
    pl.kernel,
    mesh=_mesh,
    compiler_params=pltpu.CompilerParams(needs_layout_passes=False, use_tc_tiling_on_sc=False),
    out_type=jax.ShapeDtypeStruct((E,), jnp.float32),
    scratch_types=[
        pltpu.VMEM((16,), jnp.int32),
        pltpu.VMEM((TMAX + 1, C), jnp.int32),
        pltpu.VMEM((TMAX + 1, C), jnp.int32),
        [pltpu.VMEM((C, D // 2), jnp.int32) for _ in range(NBUF)],
        [pltpu.VMEM((C, D // 2), jnp.int32) for _ in range(NBUF)],
        [pltpu.VMEM((C,), jnp.float32) for _ in range(NBUF)],
        pltpu.VMEM((256,), jnp.float32),
        [pltpu.SemaphoreType.DMA for _ in range(NBUF)],
        [pltpu.SemaphoreType.DMA for _ in range(NBUF)],
        [pltpu.SemaphoreType.DMA for _ in range(NBUF)],
        pltpu.SemaphoreType.DMA,
        pltpu.VMEM_SHARED((10000, D // 2), jnp.int32),
    ],
)
def _edge_dot(i0_hbm, i1_hbm, table_hbm, out_hbm,
              csel_v, idx0_v, idx1_v, ra, rb, ov, tp_v, sa, sb, so, si, tshr):
    wid = lax.axis_index("s") * NC + lax.axis_index("c")

    @pl.when(lax.axis_index("s") == 0)
    def _():
        pltpu.sync_copy(table_hbm, tshr)

    plsc.subcore_barrier()
    lanes = lax.iota(jnp.int32, 16)
    n_w = jnp.where(wid < NCH - TMAX * NW + NW, TMAX, TMAX - 1)

    def stage_idx(q, carry):
        csel_v[...] = jnp.minimum(wid + NW * (16 * q + lanes), NCH - 1)
        cp0 = pltpu.async_copy(
            i0_hbm.at[csel_v], idx0_v.at[pl.ds(16 * q, 16)], si)
        cp0.wait()
        cp1 = pltpu.async_copy(
            i1_hbm.at[csel_v], idx1_v.at[pl.ds(16 * q, 16)], si)
        cp1.wait()
        return carry

    lax.fori_loop(0, (TMAX + 1) // 16, stage_idx, 0)

    def issue(t, b):
        pltpu.async_copy(tshr.at[idx0_v.at[t]], ra[b], sa[b])
        pltpu.async_copy(tshr.at[idx1_v.at[t]], rb[b], sb[b])

    def compute(t, b):
        pltpu.make_async_copy(tshr.at[idx0_v.at[t]], ra[b], sa[b]).wait()
        pltpu.make_async_copy(tshr.at[idx1_v.at[t]], rb[b], sb[b]).wait()

        @pl.when(t >= NBUF)
        def _():
            pltpu.make_async_copy(
                ov[b], out_hbm.at[pl.ds(0, C)], so[b]).wait()

        hi_mask = jnp.full((16,), -65536, jnp.int32)

        def halves(w):
            h = plsc.bitcast(w & hi_mask, jnp.float32)
            l = plsc.bitcast(w << 16, jnp.float32)
            return h, l

        lanes16 = lanes << 4

        def grp(gi, carry):
            for half in range(2):
                e0 = gi * 32 + half * 16
                for j in range(16):
                    e = e0 + j
                    p = []
                    for k in range(4):
                        ah, al = halves(ra[b][e, pl.ds(16 * k, 16)])
                        bh, bl = halves(rb[b][e, pl.ds(16 * k, 16)])
                        p.append(ah * bh)
                        p.append(al * bl)
                    s01, s23 = p[0] + p[1], p[2] + p[3]
                    s45, s67 = p[4] + p[5], p[6] + p[7]
                    acc = (s01 + s23) + (s45 + s67)
                    plsc.store_scatter(tp_v, [(lanes ^ j) | (16 * j)], acc)
                cols = [plsc.load_gather(tp_v, [lanes16 | (lanes ^ c)])
                        for c in range(16)]
                while len(cols) > 1:
                    cols = [cols[i] + cols[i + 1]
                            for i in range(0, len(cols), 2)]
                ov[b][pl.ds(e0, 16)] = cols[0]
            return carry

        lax.fori_loop(0, C // 32, grp, 0)
        off = pl.multiple_of((wid + NW * t) * C, C)
        pltpu.async_copy(ov[b], out_hbm.at[pl.ds(off, C)], so[b])

    for q in range(NBUF - 1):
        issue(q, q)

    def ring(i, carry):
        g = i * NBUF
        for b in range(NBUF):
            t = g + b

            @pl.when(t + NBUF - 1 < n_w)
            def _():
                issue(t + NBUF - 1, (b + NBUF - 1) % NBUF)

            @pl.when(t < n_w)
            def _():
                compute(t, b)
        return carry

    lax.fori_loop(0, -(-TMAX // NBUF), ring, 0)

    for b in range(NBUF):
        pltpu.make_async_copy(ov[b], out_hbm.at[pl.ds(0, C)], so[b]).wait()


def kernel(model, edge_index):
    ei = edge_index.astype(jnp.int32)
    i0 = ei[0].reshape(NCH, C)
    i1 = ei[1].reshape(NCH, C)
    m16 = model.astype(jnp.bfloat16).reshape(model.shape[0], D // 2, 2)
    packed = jax.lax.bitcast_convert_type(m16, jnp.int32)
    return _edge_dot(i0, i1, packed)


# speedup vs baseline: 1.6569x; 1.6569x over previous
import functools

import jax
import jax.numpy as jnp
from jax import lax
from jax.experimental import pallas as pl
from jax.experimental.pallas import tpu as pltpu
from jax.experimental.pallas import tpu_sc as plsc

E = 320000
D = 128
NC = 2
NS = 16
NW = NC * NS
C = 128
NCH = E // C
TMAX = -(-NCH // NW)
NBUF = 2

_mesh = plsc.VectorSubcoreMesh(core_axis_name="c", subcore_axis_name="s")


@functools.partial(
    pl.kernel,
    mesh=_mesh,
    compiler_params=pltpu.CompilerParams(needs_layout_passes=False, use_tc_tiling_on_sc=False),
    out_type=jax.ShapeDtypeStruct((E,), jnp.float32),
    scratch_types=[
        pltpu.VMEM((16,), jnp.int32),
        pltpu.VMEM((TMAX + 1, C), jnp.int32),
        pltpu.VMEM((TMAX + 1, C), jnp.int32),
        [pltpu.VMEM((C, D // 2), jnp.int32) for _ in range(NBUF)],
        [pltpu.VMEM((C, D // 2), jnp.int32) for _ in range(NBUF)],
        [pltpu.VMEM((C,), jnp.float32) for _ in range(NBUF)],
        pltpu.VMEM((256,), jnp.float32),
        [pltpu.SemaphoreType.DMA for _ in range(NBUF)],
        [pltpu.SemaphoreType.DMA for _ in range(NBUF)],
        [pltpu.SemaphoreType.DMA for _ in range(NBUF)],
        pltpu.SemaphoreType.DMA,
        pltpu.VMEM_SHARED((10000, D // 2), jnp.int32),
    ],
)
def _edge_dot(i0_hbm, i1_hbm, table_hbm, out_hbm,
              csel_v, idx0_v, idx1_v, ra, rb, ov, tp_v, sa, sb, so, si, tshr):
    wid = lax.axis_index("s") * NC + lax.axis_index("c")

    @pl.when(lax.axis_index("s") == 0)
    def _():
        pltpu.sync_copy(table_hbm, tshr)

    plsc.subcore_barrier()
    lanes = lax.iota(jnp.int32, 16)
    n_w = jnp.where(wid < NCH - TMAX * NW + NW, TMAX, TMAX - 1)

    def stage_idx(q, carry):
        csel_v[...] = jnp.minimum(wid + NW * (16 * q + lanes), NCH - 1)
        cp0 = pltpu.async_copy(
            i0_hbm.at[csel_v], idx0_v.at[pl.ds(16 * q, 16)], si)
        cp0.wait()
        cp1 = pltpu.async_copy(
            i1_hbm.at[csel_v], idx1_v.at[pl.ds(16 * q, 16)], si)
        cp1.wait()
        return carry

    lax.fori_loop(0, (TMAX + 1) // 16, stage_idx, 0)

    def issue(t, b):
        pltpu.async_copy(tshr.at[idx0_v.at[t]], ra[b], sa[b])
        pltpu.async_copy(tshr.at[idx1_v.at[t]], rb[b], sb[b])

    def compute(t, b):
        pltpu.make_async_copy(tshr.at[idx0_v.at[t]], ra[b], sa[b]).wait()
        pltpu.make_async_copy(tshr.at[idx1_v.at[t]], rb[b], sb[b]).wait()

        @pl.when(t >= NBUF)
        def _():
            pltpu.make_async_copy(
                ov[b], out_hbm.at[pl.ds(0, C)], so[b]).wait()

        hi_mask = jnp.full((16,), -65536, jnp.int32)

        def halves(w):
            h = plsc.bitcast(w, jnp.float32)
            l = plsc.bitcast(w << 16, jnp.float32)
            return h, l

        def grp(gi, carry):
            for half in range(2):
                e0 = gi * 32 + half * 16
                vs = []
                for j in range(16):
                    e = e0 + j
                    p = []
                    for k in range(4):
                        ah, al = halves(ra[b][e, pl.ds(16 * k, 16)])
                        bh, bl = halves(rb[b][e, pl.ds(16 * k, 16)])
                        p.append(ah * bh)
                        p.append(al * bl)
                    s01, s23 = p[0] + p[1], p[2] + p[3]
                    s45, s67 = p[4] + p[5], p[6] + p[7]
                    s = jnp.sum((s01 + s23) + (s45 + s67))
                    vs.append(jnp.where(lanes == j, s, 0.0))
                while len(vs) > 1:
                    vs = [vs[i] + vs[i + 1] for i in range(0, len(vs), 2)]
                ov[b][pl.ds(e0, 16)] = vs[0]
            return carry

        lax.fori_loop(0, C // 32, grp, 0)
        off = pl.multiple_of((wid + NW * t) * C, C)
        pltpu.async_copy(ov[b], out_hbm.at[pl.ds(off, C)], so[b])

    for q in range(NBUF - 1):
        issue(q, q)

    def ring(i, carry):
        g = i * NBUF
        for b in range(NBUF):
            t = g + b

            @pl.when(t + NBUF - 1 < n_w)
            def _():
                issue(t + NBUF - 1, (b + NBUF - 1) % NBUF)

            @pl.when(t < n_w)
            def _():
                compute(t, b)
        return carry

    lax.fori_loop(0, -(-TMAX // NBUF), ring, 0)

    for b in range(NBUF):
        pltpu.make_async_copy(ov[b], out_hbm.at[pl.ds(0, C)], so[b]).wait()


def kernel(model, edge_index):
    ei = edge_index.astype(jnp.int32)
    i0 = ei[0].reshape(NCH, C)
    i1 = ei[1].reshape(NCH, C)
    m16 = model.astype(jnp.bfloat16).reshape(model.shape[0], D // 2, 2)
    packed = jax.lax.bitcast_convert_type(m16, jnp.int32)
    return _edge_dot(i0, i1, packed)
